# baseline (speedup 1.0000x reference)
import jax
import jax.numpy as jnp
from jax.experimental import pallas as pl
from jax.experimental.pallas import tpu as pltpu

N = 2048
BM = 256
NB = N // BM

PREC = jax.lax.Precision.DEFAULT
BF = jnp.bfloat16


def _leaky(x):
    return jnp.where(x >= 0, x, 0.01 * x)


def _dot(a, b):
    return jnp.dot(a, b, precision=PREC, preferred_element_type=jnp.float32)


def _dot16(a, b):
    return jnp.dot(a.astype(BF), b, preferred_element_type=jnp.float32)


def _body(x_ref, wp0_ref, wg0_ref, wl0_ref, wp1_ref, wg1_ref, wl1_ref,
          wp2_ref, wg2_ref, wl2_ref, adj_ref, o_ref,
          h1_scr, h2_scr, q_scr, v_scr, u_scr):
    t = pl.program_id(0)
    layer = t // NB
    i = t % NB

    def phase(h_ref, wp_ref, wg_ref, wl_ref, cin, cout, store_out, act):
        @pl.when(i == 0)
        def _prep():
            h = h_ref[...]
            q_scr[:, :cin] = _dot(h, wp_ref[...])
            v_scr[:, :cout] = _dot(h, wg_ref[...]).astype(BF)
            u_scr[:, :cout] = _dot(h, wl_ref[...]).astype(BF)

        q_i = q_scr[pl.ds(i * BM, BM), :cin]
        logits = jax.lax.dot_general(
            q_i, h_ref[...], (((1,), (1,)), ((), ())),
            precision=PREC, preferred_element_type=jnp.float32)
        m = jnp.max(logits, axis=1, keepdims=True)
        e = jnp.exp(logits - m)
        s = jnp.sum(e, axis=1, keepdims=True)
        og = _dot16(e, v_scr[:, :cout]) / s
        ol = _dot16(adj_ref[...], u_scr[:, :cout])
        if act:
            out = _leaky(og) + _leaky(ol)
        else:
            out = og + ol
        store_out(out)

    @pl.when(layer == 0)
    def _l0():
        def store(out):
            h1_scr[pl.ds(i * BM, BM), :] = out
        phase(x_ref, wp0_ref, wg0_ref, wl0_ref, 256, 256, store, True)

    @pl.when(layer == 1)
    def _l1():
        def store(out):
            h2_scr[pl.ds(i * BM, BM), :] = out
        phase(h1_scr, wp1_ref, wg1_ref, wl1_ref, 256, 512, store, True)

    @pl.when(layer == 2)
    def _l2():
        def store(out):
            o_ref[pl.ds(i * BM, BM), :] = out
        phase(h2_scr, wp2_ref, wg2_ref, wl2_ref, 512, 64, store, False)


def kernel(x, adj, Wp0, Wg0, Wl0, Wp1, Wg1, Wl1, Wp2, Wg2, Wl2):
    f32 = jnp.float32
    return pl.pallas_call(
        _body,
        grid=(3 * NB,),
        in_specs=[
            pl.BlockSpec((N, 256), lambda t: (0, 0)),
            pl.BlockSpec((256, 256), lambda t: (0, 0)),
            pl.BlockSpec((256, 256), lambda t: (0, 0)),
            pl.BlockSpec((256, 256), lambda t: (0, 0)),
            pl.BlockSpec((256, 256), lambda t: (0, 0)),
            pl.BlockSpec((256, 512), lambda t: (0, 0)),
            pl.BlockSpec((256, 512), lambda t: (0, 0)),
            pl.BlockSpec((512, 512), lambda t: (0, 0)),
            pl.BlockSpec((512, 64), lambda t: (0, 0)),
            pl.BlockSpec((512, 64), lambda t: (0, 0)),
            pl.BlockSpec((BM, N), lambda t: (t % NB, 0)),
        ],
        out_specs=pl.BlockSpec((N, 64), lambda t: (0, 0)),
        out_shape=jax.ShapeDtypeStruct((N, 64), f32),
        scratch_shapes=[
            pltpu.VMEM((N, 256), f32),
            pltpu.VMEM((N, 512), f32),
            pltpu.VMEM((N, 512), f32),
            pltpu.VMEM((N, 512), BF),
            pltpu.VMEM((N, 512), BF),
        ],
    )(x, Wp0, Wg0, Wl0, Wp1, Wg1, Wl1, Wp2, Wg2, Wl2, adj)
